# baseline (speedup 1.0000x reference)
import functools

import jax
import jax.numpy as jnp
from jax import lax
from jax.experimental import pallas as pl
from jax.experimental.pallas import tpu as pltpu
from jax.experimental.pallas import tpu_sc as plsc

BATCH = 16384
SEQ = 50
DIM = 65
PAD_DIM = 128
NUM_ROWS = BATCH * SEQ
NUM_WORKERS = 32
QUADS = NUM_ROWS // 200
Q_PER_W = QUADS // NUM_WORKERS
GROUP = 200
HALF = 100

PAD_BLK = 2000


def _pad_kernel(w_ref, o_ref):
    o_ref[:, :DIM] = w_ref[...]


def _tc_pad(weight):
    return pl.pallas_call(
        _pad_kernel,
        grid=(weight.shape[0] // PAD_BLK,),
        in_specs=[pl.BlockSpec((PAD_BLK, DIM), lambda i: (i, 0))],
        out_specs=pl.BlockSpec((PAD_BLK, PAD_DIM), lambda i: (i, 0)),
        out_shape=jax.ShapeDtypeStruct((weight.shape[0], PAD_DIM), jnp.float32),
    )(weight)


def _gather_kernel(idx_hbm, table_hbm, out_hbm, idx_v, rows0, rows1, sem):
    wid = lax.axis_index("s") * 2 + lax.axis_index("c")
    q0 = wid * Q_PER_W
    pltpu.sync_copy(idx_hbm.at[pl.ds(q0, Q_PER_W)], idx_v)
    bufs = (rows0, rows1)

    def gather(q, buf):
        pltpu.async_copy(table_hbm.at[idx_v.at[q].at[pl.ds(0, HALF)]],
                         buf.at[pl.ds(0, HALF)], sem)
        pltpu.async_copy(table_hbm.at[idx_v.at[q].at[pl.ds(PAD_DIM, HALF)]],
                         buf.at[pl.ds(HALF, HALF)], sem)

    def drain(buf):
        pltpu.make_async_copy(table_hbm.at[idx_v.at[0].at[pl.ds(0, HALF)]],
                              buf.at[pl.ds(0, HALF)], sem).wait()
        pltpu.make_async_copy(table_hbm.at[idx_v.at[0].at[pl.ds(0, HALF)]],
                              buf.at[pl.ds(HALF, HALF)], sem).wait()

    gather(0, rows0)

    def body(q2, _):
        for k in range(2):
            q = 2 * q2 + k
            buf = bufs[k]

            @pl.when(q + 1 < Q_PER_W)
            def _():
                gather(q + 1, bufs[1 - k])

            drain(buf)
            pltpu.sync_copy(buf, out_hbm.at[pl.ds((q0 + q) * GROUP, GROUP)])
        return 0

    lax.fori_loop(0, Q_PER_W // 2, body, 0)


def _sc_gather(idx, table):
    mesh = plsc.VectorSubcoreMesh(core_axis_name="c", subcore_axis_name="s")
    k = functools.partial(
        pl.kernel,
        mesh=mesh,
        out_type=jax.ShapeDtypeStruct((NUM_ROWS, PAD_DIM), jnp.float32),
        scratch_types=[
            pltpu.VMEM((Q_PER_W, 2 * PAD_DIM), jnp.int32),
            pltpu.VMEM((GROUP, PAD_DIM), jnp.float32),
            pltpu.VMEM((GROUP, PAD_DIM), jnp.float32),
            pltpu.SemaphoreType.DMA,
        ],
    )(_gather_kernel)
    return k(idx, table)


def kernel(indices, weight):
    table = lax.dynamic_update_slice(
        jnp.zeros((weight.shape[0], PAD_DIM), jnp.float32),
        weight.astype(jnp.float32), (0, 0))
    idx = jnp.pad(indices.reshape(2 * QUADS, HALF).astype(jnp.int32),
                  ((0, 0), (0, PAD_DIM - HALF))).reshape(QUADS, 2 * PAD_DIM)
    slab = _sc_gather(idx, table)
    return slab[:, :DIM].reshape(BATCH, SEQ, DIM)


# speedup vs baseline: 1.2663x; 1.2663x over previous
import functools

import jax
import jax.numpy as jnp
from jax import lax
from jax.experimental import pallas as pl
from jax.experimental.pallas import tpu as pltpu
from jax.experimental.pallas import tpu_sc as plsc

BATCH = 16384
SEQ = 50
DIM = 65
PAD_DIM = 128
NUM_ROWS = BATCH * SEQ
NUM_WORKERS = 32
GROUP = 256
HALF = 128
GROUPS = NUM_ROWS // GROUP
Q_PER_W = GROUPS // NUM_WORKERS

PAD_BLK = 2000


def _pad_kernel(w_ref, o_ref):
    o_ref[:, :DIM] = w_ref[...]


def _tc_pad(weight):
    return pl.pallas_call(
        _pad_kernel,
        grid=(weight.shape[0] // PAD_BLK,),
        in_specs=[pl.BlockSpec((PAD_BLK, DIM), lambda i: (i, 0))],
        out_specs=pl.BlockSpec((PAD_BLK, PAD_DIM), lambda i: (i, 0)),
        out_shape=jax.ShapeDtypeStruct((weight.shape[0], PAD_DIM), jnp.float32),
    )(weight)


def _gather_kernel(idx_hbm, table_hbm, out_hbm, idx_v, rows0, rows1, sem):
    wid = lax.axis_index("s") * 2 + lax.axis_index("c")
    q0 = wid * Q_PER_W
    pltpu.sync_copy(idx_hbm.at[pl.ds(2 * q0, 2 * Q_PER_W)], idx_v)
    bufs = (rows0, rows1)

    def gather(q, buf):
        pltpu.async_copy(table_hbm.at[idx_v.at[2 * q]],
                         buf.at[pl.ds(0, HALF)], sem)
        pltpu.async_copy(table_hbm.at[idx_v.at[2 * q + 1]],
                         buf.at[pl.ds(HALF, HALF)], sem)

    def drain(buf):
        pltpu.make_async_copy(table_hbm.at[idx_v.at[0]],
                              buf.at[pl.ds(0, HALF)], sem).wait()
        pltpu.make_async_copy(table_hbm.at[idx_v.at[0]],
                              buf.at[pl.ds(HALF, HALF)], sem).wait()

    gather(0, rows0)

    def body(q2, _):
        for k in range(2):
            q = 2 * q2 + k
            buf = bufs[k]

            @pl.when(q + 1 < Q_PER_W)
            def _():
                gather(q + 1, bufs[1 - k])

            drain(buf)
            pltpu.sync_copy(buf, out_hbm.at[pl.ds((q0 + q) * GROUP, GROUP)])
        return 0

    lax.fori_loop(0, Q_PER_W // 2, body, 0)


def _sc_gather(idx, table):
    mesh = plsc.VectorSubcoreMesh(core_axis_name="c", subcore_axis_name="s")
    k = functools.partial(
        pl.kernel,
        mesh=mesh,
        out_type=jax.ShapeDtypeStruct((NUM_ROWS, PAD_DIM), jnp.float32),
        scratch_types=[
            pltpu.VMEM((2 * Q_PER_W, HALF), jnp.int32),
            pltpu.VMEM((GROUP, PAD_DIM), jnp.float32),
            pltpu.VMEM((GROUP, PAD_DIM), jnp.float32),
            pltpu.SemaphoreType.DMA,
        ],
    )(_gather_kernel)
    return k(idx, table)


def kernel(indices, weight):
    table = _tc_pad(weight.astype(jnp.float32))
    idx = indices.reshape(2 * GROUPS, HALF).astype(jnp.int32)
    slab = _sc_gather(idx, table)
    return slab[:, :DIM].reshape(BATCH, SEQ, DIM)
